# baseline (speedup 1.0000x reference)
import functools

import numpy as np
import jax
import jax.numpy as jnp
from jax import lax
from jax.experimental import pallas as pl
from jax.experimental.pallas import tpu as pltpu
from jax.experimental.pallas import tpu_sc as plsc

_B, _C, _H, _W = 4, 96, 384, 384
_N = _B * _H * _W
_P = 128
_NCHUNK = _W // _P

_info = plsc.get_sparse_core_info()
_NC, _NS, _L = _info.num_cores, _info.num_subcores, _info.num_lanes
_NW = _NC * _NS
_ROWS_PER_W = (_B * _H) // _NW
_CHUNKS_PER_W = _ROWS_PER_W * _NCHUNK

_F1 = np.float32(1.0)
_F0 = np.float32(0.0)


def _vec_weights(coord, lim):
    t = coord.astype(jnp.int32)
    x0 = jnp.where(t.astype(jnp.float32) > coord, t - 1, t)
    fx = coord - x0.astype(jnp.float32)
    inb = (x0 >= 0) & (x0 <= lim - 2)
    wa = jnp.where(x0 == -1, fx, jnp.where(inb, _F1 - fx, _F0))
    wb = jnp.where(x0 == lim - 1, _F1 - fx, jnp.where(inb, fx, _F0))
    return wa, wb


def _warp_body(table_hbm, coef_hbm, out_hbm, coef_v, idx_v, w_v, g_v, o_v,
               sem_g, sem_o):
    wid = lax.axis_index("s") * _NC + lax.axis_index("c")
    pltpu.sync_copy(coef_hbm, coef_v)
    iota = lax.broadcasted_iota(jnp.int32, (_L,), 0)

    b = (wid * _ROWS_PER_W) // _H
    ax = coef_v[b, 0, :]
    bx = coef_v[b, 1, :]
    cx = coef_v[b, 2, :]
    ay = coef_v[b, 3, :]
    by = coef_v[b, 4, :]
    cy = coef_v[b, 5, :]
    bhw = b * (_H * _W)

    def idx_fire(k):
        buf = k % 2
        row = wid * _ROWS_PER_W + k // _NCHUNK
        ck = k % _NCHUNK
        yf = (row - b * _H).astype(jnp.float32)
        rx = bx * yf + cx
        ry = by * yf + cy
        for g in range(_P // _L):
            pix = g * _L + iota
            xf = (pix + ck * _P).astype(jnp.float32)
            ix = ax * xf + rx
            iy = ay * xf + ry
            wa, wb = _vec_weights(ix, _W)
            va, vb = _vec_weights(iy, _H)
            xs = jnp.clip(ix.astype(jnp.int32), 0, _W - 2)
            ys = jnp.clip(iy.astype(jnp.int32), 0, _H - 2)
            base = bhw + ys * _W + xs
            sl = pl.ds(g * _L, _L)
            idx_v[buf, 0, sl] = base
            idx_v[buf, 1, sl] = base + 1
            idx_v[buf, 2, sl] = base + _W
            idx_v[buf, 3, sl] = base + _W + 1
            wq = pix * 4
            plsc.store_scatter(w_v.at[buf], [wq], wa * va)
            plsc.store_scatter(w_v.at[buf], [wq + 1], wb * va)
            plsc.store_scatter(w_v.at[buf], [wq + 2], wa * vb)
            plsc.store_scatter(w_v.at[buf], [wq + 3], wb * vb)
        for t in range(4):
            pltpu.async_copy(
                table_hbm.at[idx_v.at[buf, t]], g_v.at[buf, t], sem_g
            )

    def wait_gathers(k):
        buf = k % 2
        for t in range(4):
            pltpu.make_async_copy(
                table_hbm.at[idx_v.at[buf, t]], g_v.at[buf, t], sem_g
            ).wait()

    def drain_one_row_store():
        pltpu.make_async_copy(
            o_v.at[0], out_hbm.at[pl.ds(0, _C), pl.ds(0, _W)], sem_o
        ).wait()

    def blend_store(k):
        buf = k % 2
        row = wid * _ROWS_PER_W + k // _NCHUNK
        rslot = (k // _NCHUNK) % 2
        ck = k % _NCHUNK
        y = row - b * _H

        @pl.when((ck == 0) & (k // _NCHUNK >= 2))
        def _():
            drain_one_row_store()

        @plsc.parallel_loop(0, _P, 1, unroll=4)
        def blend(p):
            wv = w_v[buf, pl.ds(p * 4, _L)]
            w0 = wv[0]
            w1 = wv[1]
            w2 = wv[2]
            w3 = wv[3]
            x = ck * _P + p
            xsp = jnp.full((_L,), x, jnp.int32)
            for j in range(_C // (2 * _L)):
                sj = pl.ds(j * 2 * _L, 2 * _L)
                e0, o0 = plsc.unpack(
                    g_v[buf, 0, p, sj], format=plsc.PackFormat.INTERLEAVED
                )
                e1, o1 = plsc.unpack(
                    g_v[buf, 1, p, sj], format=plsc.PackFormat.INTERLEAVED
                )
                e2, o2 = plsc.unpack(
                    g_v[buf, 2, p, sj], format=plsc.PackFormat.INTERLEAVED
                )
                e3, o3 = plsc.unpack(
                    g_v[buf, 3, p, sj], format=plsc.PackFormat.INTERLEAVED
                )
                ve = w0 * e0 + w1 * e1 + w2 * e2 + w3 * e3
                vo = w0 * o0 + w1 * o1 + w2 * o2 + w3 * o3
                che = j * 2 * _L + 2 * iota
                plsc.store_scatter(o_v.at[rslot], [che, xsp], ve)
                plsc.store_scatter(o_v.at[rslot], [che + 1, xsp], vo)

        @pl.when(ck == _NCHUNK - 1)
        def _():
            pltpu.async_copy(
                o_v.at[rslot],
                out_hbm.at[pl.ds(b * _C, _C), pl.ds(y * _W, _W)],
                sem_o,
            )

    def step(i, carry):
        idx_fire(i)

        @pl.when(i > 0)
        def _():
            wait_gathers(i - 1)
            blend_store(i - 1)

        return carry

    lax.fori_loop(0, _CHUNKS_PER_W, step, 0)
    wait_gathers(_CHUNKS_PER_W - 1)
    blend_store(_CHUNKS_PER_W - 1)
    drain_one_row_store()
    drain_one_row_store()


_warp_sc = functools.partial(
    pl.kernel,
    out_type=jax.ShapeDtypeStruct((_B * _C, _H * _W), jnp.float32),
    mesh=plsc.VectorSubcoreMesh(core_axis_name="c", subcore_axis_name="s"),
    compiler_params=pltpu.CompilerParams(
        use_tc_tiling_on_sc=False, needs_layout_passes=False
    ),
    scratch_types=[
        pltpu.VMEM((_B, 6, _L), jnp.float32),
        pltpu.VMEM((2, 4, _P), jnp.int32),
        pltpu.VMEM((2, 4 * _P + _L), jnp.float32),
        pltpu.VMEM((2, 4, _P, _C), jnp.bfloat16),
        pltpu.VMEM((2, _C, _W), jnp.float32),
        pltpu.SemaphoreType.DMA,
        pltpu.SemaphoreType.DMA,
    ],
)(_warp_body)


def kernel(input_tensors, shift_x, shift_y, scale, rotation_angle):
    B, C, H, W = input_tensors.shape
    f32 = jnp.float32
    s = scale[:, 0, 0].astype(f32)
    sx = shift_x[:, 0, 0].astype(f32)
    sy = shift_y[:, 0, 0].astype(f32)
    ang = rotation_angle[:, 0, 0].astype(f32)
    inv_s = 1.0 / s
    half_w = ((W - 1 + sx[0]) * inv_s[0] - sx[0] * inv_s[0]) / 2.0
    half_h = ((H - 1 + sy[0]) * inv_s[0] - sy[0] * inv_s[0]) / 2.0
    th = ang * f32(np.pi / 180.0)
    c = jnp.cos(th)
    si = jnp.sin(th)
    cxb = sx * inv_s - half_w
    cyb = sy * inv_s - half_h
    kx = f32(W / (W - 1))
    ky = f32(H / (H - 1))
    coefs = jnp.stack(
        [
            c * inv_s * kx,
            -si * inv_s * kx,
            (c * cxb - si * cyb + half_w) * kx - 0.5,
            si * inv_s * ky,
            c * inv_s * ky,
            (si * cxb + c * cyb + half_h) * ky - 0.5,
        ],
        axis=-1,
    ).astype(f32)
    coefs = jnp.broadcast_to(coefs[:, :, None], (B, 6, 16))

    table = (
        input_tensors.astype(jnp.bfloat16)
        .transpose(0, 2, 3, 1)
        .reshape(B * H * W, C)
    )
    out = _warp_sc(table, coefs)
    return out.reshape(B, C, H, W)


# speedup vs baseline: 1.4406x; 1.4406x over previous
import functools

import numpy as np
import jax
import jax.numpy as jnp
from jax import lax
from jax.experimental import pallas as pl
from jax.experimental.pallas import tpu as pltpu
from jax.experimental.pallas import tpu_sc as plsc

_B, _C, _H, _W = 4, 96, 384, 384
_N = _B * _H * _W
_P = 128
_NCHUNK = _W // _P

_info = plsc.get_sparse_core_info()
_NC, _NS, _L = _info.num_cores, _info.num_subcores, _info.num_lanes
_NW = _NC * _NS
_ROWS_PER_W = (_B * _H) // _NW
_CHUNKS_PER_W = _ROWS_PER_W * _NCHUNK

_F1 = np.float32(1.0)
_F0 = np.float32(0.0)


def _vec_weights(coord, lim):
    t = coord.astype(jnp.int32)
    x0 = jnp.where(t.astype(jnp.float32) > coord, t - 1, t)
    fx = coord - x0.astype(jnp.float32)
    inb = (x0 >= 0) & (x0 <= lim - 2)
    wa = jnp.where(x0 == -1, fx, jnp.where(inb, _F1 - fx, _F0))
    wb = jnp.where(x0 == lim - 1, _F1 - fx, jnp.where(inb, fx, _F0))
    return wa, wb


def _warp_body(table_hbm, coef_hbm, out_hbm, coef_v, idx_v, w_v, g_v, o_v,
               sem_g, sem_o):
    wid = lax.axis_index("s") * _NC + lax.axis_index("c")
    pltpu.sync_copy(coef_hbm, coef_v)
    iota = lax.broadcasted_iota(jnp.int32, (_L,), 0)

    b = (wid * _ROWS_PER_W) // _H
    ax = coef_v[b, 0, :]
    bx = coef_v[b, 1, :]
    cx = coef_v[b, 2, :]
    ay = coef_v[b, 3, :]
    by = coef_v[b, 4, :]
    cy = coef_v[b, 5, :]
    bhw = b * (_H * _W)

    def idx_fire(k):
        buf = k % 2
        row = wid * _ROWS_PER_W + k // _NCHUNK
        ck = k % _NCHUNK
        yf = (row - b * _H).astype(jnp.float32)
        rx = bx * yf + cx
        ry = by * yf + cy
        for g in range(_P // _L):
            pix = g * _L + iota
            xf = (pix + ck * _P).astype(jnp.float32)
            ix = ax * xf + rx
            iy = ay * xf + ry
            wa, wb = _vec_weights(ix, _W)
            va, vb = _vec_weights(iy, _H)
            xs = jnp.clip(ix.astype(jnp.int32), 0, _W - 2)
            ys = jnp.clip(iy.astype(jnp.int32), 0, _H - 2)
            base = bhw + ys * _W + xs
            sl = pl.ds(g * _L, _L)
            idx_v[buf, 0, sl] = base
            idx_v[buf, 1, sl] = base + 1
            idx_v[buf, 2, sl] = base + _W
            idx_v[buf, 3, sl] = base + _W + 1
            wq = pix * 4
            plsc.store_scatter(w_v.at[buf], [wq], wa * va)
            plsc.store_scatter(w_v.at[buf], [wq + 1], wb * va)
            plsc.store_scatter(w_v.at[buf], [wq + 2], wa * vb)
            plsc.store_scatter(w_v.at[buf], [wq + 3], wb * vb)
        for t in range(4):
            pltpu.async_copy(
                table_hbm.at[idx_v.at[buf, t]], g_v.at[buf, t], sem_g
            )

    def wait_gathers(k):
        buf = k % 2
        for t in range(4):
            pltpu.make_async_copy(
                table_hbm.at[idx_v.at[buf, t]], g_v.at[buf, t], sem_g
            ).wait()

    def drain_one_store():
        pltpu.make_async_copy(
            o_v.at[0], out_hbm.at[pl.ds(0, _P)], sem_o
        ).wait()

    def blend_store(k):
        buf = k % 2
        row = wid * _ROWS_PER_W + k // _NCHUNK
        pix0 = row * _W + (k % _NCHUNK) * _P

        @pl.when(k >= 2)
        def _():
            drain_one_store()

        @plsc.parallel_loop(0, _P, 1, unroll=4)
        def blend(p):
            wv = w_v[buf, pl.ds(p * 4, _L)]
            w0 = wv[0]
            w1 = wv[1]
            w2 = wv[2]
            w3 = wv[3]
            for j in range(_C // _L):
                sj = pl.ds(j * _L, _L)
                o_v[buf, p, sj] = (
                    w0 * g_v[buf, 0, p, sj]
                    + w1 * g_v[buf, 1, p, sj]
                    + w2 * g_v[buf, 2, p, sj]
                    + w3 * g_v[buf, 3, p, sj]
                )

        pltpu.async_copy(o_v.at[buf], out_hbm.at[pl.ds(pix0, _P)], sem_o)

    def step(i, carry):
        idx_fire(i)

        @pl.when(i > 0)
        def _():
            wait_gathers(i - 1)
            blend_store(i - 1)

        return carry

    lax.fori_loop(0, _CHUNKS_PER_W, step, 0)
    wait_gathers(_CHUNKS_PER_W - 1)
    blend_store(_CHUNKS_PER_W - 1)
    drain_one_store()
    drain_one_store()


_warp_sc = functools.partial(
    pl.kernel,
    out_type=jax.ShapeDtypeStruct((_N, _C), jnp.float32),
    mesh=plsc.VectorSubcoreMesh(core_axis_name="c", subcore_axis_name="s"),
    compiler_params=pltpu.CompilerParams(
        use_tc_tiling_on_sc=False, needs_layout_passes=False
    ),
    scratch_types=[
        pltpu.VMEM((_B, 6, _L), jnp.float32),
        pltpu.VMEM((2, 4, _P), jnp.int32),
        pltpu.VMEM((2, 4 * _P + _L), jnp.float32),
        pltpu.VMEM((2, 4, _P, _C), jnp.float32),
        pltpu.VMEM((2, _P, _C), jnp.float32),
        pltpu.SemaphoreType.DMA,
        pltpu.SemaphoreType.DMA,
    ],
)(_warp_body)


def kernel(input_tensors, shift_x, shift_y, scale, rotation_angle):
    B, C, H, W = input_tensors.shape
    f32 = jnp.float32
    s = scale[:, 0, 0].astype(f32)
    sx = shift_x[:, 0, 0].astype(f32)
    sy = shift_y[:, 0, 0].astype(f32)
    ang = rotation_angle[:, 0, 0].astype(f32)
    inv_s = 1.0 / s
    half_w = ((W - 1 + sx[0]) * inv_s[0] - sx[0] * inv_s[0]) / 2.0
    half_h = ((H - 1 + sy[0]) * inv_s[0] - sy[0] * inv_s[0]) / 2.0
    th = ang * f32(np.pi / 180.0)
    c = jnp.cos(th)
    si = jnp.sin(th)
    cxb = sx * inv_s - half_w
    cyb = sy * inv_s - half_h
    kx = f32(W / (W - 1))
    ky = f32(H / (H - 1))
    coefs = jnp.stack(
        [
            c * inv_s * kx,
            -si * inv_s * kx,
            (c * cxb - si * cyb + half_w) * kx - 0.5,
            si * inv_s * ky,
            c * inv_s * ky,
            (si * cxb + c * cyb + half_h) * ky - 0.5,
        ],
        axis=-1,
    ).astype(f32)
    coefs = jnp.broadcast_to(coefs[:, :, None], (B, 6, 16))

    table = input_tensors.transpose(0, 2, 3, 1).reshape(B * H * W, C)
    out = _warp_sc(table, coefs)
    return out.reshape(B, H, W, C).transpose(0, 3, 1, 2)
